# baseline (speedup 1.0000x reference)
import functools

import jax
import jax.numpy as jnp
from jax import lax
from jax.experimental import pallas as pl
from jax.experimental.pallas import tpu as pltpu
from jax.experimental.pallas import tpu_sc as plsc

NC = 2
NS = 16
L = 16
NW = NC * NS

K = 8
R = 129
NI = R * K
MAGIC = 0x5F3759DF


def _rsqrt_newton(s):
    i = plsc.bitcast(s, jnp.int32)
    i = jnp.int32(MAGIC) - (i >> 1)
    y = plsc.bitcast(i, jnp.float32)
    for _ in range(3):
        y = y * (1.5 - 0.5 * s * y * y)
    return y


def _sc_body(N, Dm1, NBw, NBp,
             ids_hbm, pos_hbm, tid_hbm, word_hbm, posemb_hbm, type_hbm,
             curv_hbm, out_hbm,
             ids_s, pos_s, tid_s, phw_v, php_v, idxw_v, idxp_v,
             gw_v, gp_v, ty_v, dty_v, obuf, curv_v, semw, semp):
    D = Dm1 + 1
    tpw = N // NW
    nchunk = tpw // K
    njc = D // L
    wid = lax.axis_index("s") * NC + lax.axis_index("c")
    base = wid * tpw

    iota = lax.iota(jnp.int32, L)
    lo8 = iota & 7
    hi8 = jnp.where(iota >= 8, 1, 0)

    pltpu.sync_copy(curv_hbm, curv_v)
    inv_c = 1.0 / curv_v[...]

    pltpu.sync_copy(type_hbm, ty_v.at[pl.ds(0, 2 * Dm1)])
    for jj in range(njc):
        cidx = jnp.minimum(jj * L + iota, Dm1 - 1)
        d = (plsc.load_gather(ty_v, [Dm1 + cidx])
             - plsc.load_gather(ty_v, [cidx]))
        dty_v[pl.ds(jj * L, L)] = d

    def chunk_body(c, _):
        off = base + c * K
        pltpu.sync_copy(ids_hbm.at[pl.ds(off, K)], ids_s.at[pl.ds(0, K)])
        pltpu.sync_copy(pos_hbm.at[pl.ds(off, K)], pos_s.at[pl.ds(0, K)])
        pltpu.sync_copy(tid_hbm.at[pl.ds(off, K)], tid_s.at[pl.ds(0, K)])

        idsd = plsc.load_gather(ids_s, [lo8])
        posd = plsc.load_gather(pos_s, [lo8])
        tid_s[...] = plsc.load_gather(tid_s, [lo8])
        sw = idsd * Dm1
        sp = posd * Dm1
        b0w = sw >> 4
        b0p = sp >> 4
        phw_v[...] = sw & 15
        php_v[...] = sp & 15
        for i in range(64):
            idxw_v[pl.ds(16 * i, L)] = b0w + (2 * i + hi8)
            idxp_v[pl.ds(16 * i, L)] = b0p + (2 * i + hi8)
        plsc.store_scatter(idxw_v, [1024 + iota],
                           jnp.minimum(b0w + 128, NBw - 1), mask=iota < 8)
        plsc.store_scatter(idxp_v, [1024 + iota],
                           jnp.minimum(b0p + 128, NBp - 1), mask=iota < 8)

        cw = pltpu.async_copy(word_hbm.at[idxw_v], gw_v, semw)
        cp = pltpu.async_copy(posemb_hbm.at[idxp_v], gp_v, semp)
        cw.wait()
        cp.wait()

        for t in range(K):
            t16 = jnp.full((L,), t + 8, jnp.int32)
            phw = plsc.load_gather(phw_v, [t16])
            php = plsc.load_gather(php_v, [t16])
            tidf = plsc.load_gather(tid_s, [t16]).astype(jnp.float32)
            aw = phw - 1 + iota
            ap = php - 1 + iota
            rw = 8 * (aw >> 4) + t
            lw = aw & 15
            rp = 8 * (ap >> 4) + t
            lp = ap & 15

            def col_body(j, sumsq):
                j8 = j * 8
                cvec = j * L - 1 + iota
                ty0c = plsc.load_gather(ty_v, [cvec])
                dtyc = plsc.load_gather(dty_v, [cvec])
                x = (plsc.load_gather(gw_v, [rw + j8, lw])
                     + plsc.load_gather(gp_v, [rp + j8, lp]))
                x = x + ty0c + tidf * dtyc
                obuf[t, pl.ds(j * L, L)] = x
                return sumsq + x * x

            sumsq = lax.fori_loop(1, njc, col_body,
                                  jnp.zeros((L,), jnp.float32))
            c0 = jnp.maximum(iota - 1, 0)
            q0w = phw + c0
            q0p = php + c0
            x0 = (plsc.load_gather(gw_v, [8 * (q0w >> 4) + t, q0w & 15])
                  + plsc.load_gather(gp_v, [8 * (q0p >> 4) + t, q0p & 15]))
            x0 = (x0 + plsc.load_gather(ty_v, [c0])
                  + tidf * plsc.load_gather(dty_v, [c0]))
            sumsq = sumsq + jnp.where(iota >= 1, x0 * x0, 0.0)
            s = inv_c + jnp.sum(sumsq)
            time = s * _rsqrt_newton(s)
            obuf[t, pl.ds(0, L)] = jnp.where(iota == 0, time, x0)

        pltpu.sync_copy(obuf, out_hbm.at[pl.ds(off, K)])
        return 0

    lax.fori_loop(0, nchunk, chunk_body, 0)


def kernel(input_ids, position_ids, token_type_ids, word_emb, pos_emb,
           type_emb, curvature):
    B, S = input_ids.shape
    V, Dm1 = word_emb.shape
    MP = pos_emb.shape[0]
    N = B * S
    D = Dm1 + 1
    NBw = V * Dm1 // L
    NBp = MP * Dm1 // L

    ids = input_ids.reshape(-1).astype(jnp.int32)
    pos = position_ids.reshape(-1).astype(jnp.int32)
    tid = token_type_ids.reshape(-1).astype(jnp.int32)
    wblk = word_emb.reshape(NBw, L)
    pblk = pos_emb.reshape(NBp, L)
    tyflat = type_emb.reshape(-1)
    curv16 = jnp.full((L,), curvature, jnp.float32)

    mesh = plsc.VectorSubcoreMesh(core_axis_name="c", subcore_axis_name="s",
                                  num_cores=NC, num_subcores=NS)
    body = functools.partial(_sc_body, N, Dm1, NBw, NBp)
    sck = pl.kernel(
        body,
        out_type=jax.ShapeDtypeStruct((N, D), jnp.float32),
        mesh=mesh,
        compiler_params=pltpu.CompilerParams(use_tc_tiling_on_sc=False,
                                             needs_layout_passes=False),
        scratch_types=[
            pltpu.VMEM((L,), jnp.int32),
            pltpu.VMEM((L,), jnp.int32),
            pltpu.VMEM((L,), jnp.int32),
            pltpu.VMEM((L,), jnp.int32),
            pltpu.VMEM((L,), jnp.int32),
            pltpu.VMEM((NI,), jnp.int32),
            pltpu.VMEM((NI,), jnp.int32),
            pltpu.VMEM((NI, L), jnp.float32),
            pltpu.VMEM((NI, L), jnp.float32),
            pltpu.VMEM((2 * D, ), jnp.float32),
            pltpu.VMEM((D,), jnp.float32),
            pltpu.VMEM((K, D), jnp.float32),
            pltpu.VMEM((L,), jnp.float32),
            pltpu.SemaphoreType.DMA,
            pltpu.SemaphoreType.DMA,
        ],
    )
    out = sck(ids, pos, tid, wblk, pblk, tyflat, curv16)
    return out.reshape(B, S, D)


# speedup vs baseline: 1.2394x; 1.2394x over previous
import functools

import jax
import jax.numpy as jnp
from jax import lax
from jax.experimental import pallas as pl
from jax.experimental.pallas import tpu as pltpu
from jax.experimental.pallas import tpu_sc as plsc

NC = 2
NS = 16
L = 16
NW = NC * NS

K = 8
R = 129
NI = R * K
MAGIC = 0x5F3759DF


def _rsqrt_newton(s):
    i = plsc.bitcast(s, jnp.int32)
    i = jnp.int32(MAGIC) - (i >> 1)
    y = plsc.bitcast(i, jnp.float32)
    for _ in range(3):
        y = y * (1.5 - 0.5 * s * y * y)
    return y


def _sc_body(N, Dm1, NBw, NBp,
             ids_hbm, pos_hbm, tid_hbm, word_hbm, posemb_hbm, type_hbm,
             curv_hbm, out_hbm,
             ids_v, pos_v, tidq_v, tid_s, phw_v, php_v, idxw_v, idxp_v,
             gw_v, gp_v, ty_v, dty_v, obuf, curv_v, semw, semp):
    D = Dm1 + 1
    tpw = N // NW
    nchunk = tpw // K
    npair = nchunk // 2
    njc = D // L
    wid = lax.axis_index("s") * NC + lax.axis_index("c")
    base = wid * tpw

    iota = lax.iota(jnp.int32, L)
    lo8 = iota & 7
    hi8 = jnp.where(iota >= 8, 1, 0)

    pltpu.sync_copy(curv_hbm, curv_v)
    inv_c = 1.0 / curv_v[...]

    pltpu.sync_copy(ids_hbm.at[pl.ds(base, tpw)], ids_v)
    pltpu.sync_copy(pos_hbm.at[pl.ds(base, tpw)], pos_v)
    pltpu.sync_copy(tid_hbm.at[pl.ds(base, tpw)], tidq_v)

    pltpu.sync_copy(type_hbm, ty_v.at[pl.ds(0, 2 * Dm1)])
    for jj in range(njc):
        cidx = jnp.minimum(jj * L + iota, Dm1 - 1)
        d = (plsc.load_gather(ty_v, [Dm1 + cidx])
             - plsc.load_gather(ty_v, [cidx]))
        dty_v[pl.ds(jj * L, L)] = d

    def launch(c, buf):
        idxw, idxp, phw_b, php_b = idxw_v.at[buf], idxp_v.at[buf], phw_v.at[buf], php_v.at[buf]
        sel = c * K + lo8
        idsd = plsc.load_gather(ids_v, [sel])
        posd = plsc.load_gather(pos_v, [sel])
        sw = idsd * Dm1
        sp = posd * Dm1
        b0w = sw >> 4
        b0p = sp >> 4
        phw_b[...] = sw & 15
        php_b[...] = sp & 15
        for i in range(64):
            idxw[pl.ds(16 * i, L)] = b0w + (2 * i + hi8)
            idxp[pl.ds(16 * i, L)] = b0p + (2 * i + hi8)
        plsc.store_scatter(idxw, [1024 + iota],
                           jnp.minimum(b0w + 128, NBw - 1), mask=iota < 8)
        plsc.store_scatter(idxp, [1024 + iota],
                           jnp.minimum(b0p + 128, NBp - 1), mask=iota < 8)
        pltpu.async_copy(word_hbm.at[idxw], gw_v.at[buf], semw.at[buf])
        pltpu.async_copy(posemb_hbm.at[idxp], gp_v.at[buf], semp.at[buf])

    def compute(c, buf):
        gw, gp, phw_b, php_b = gw_v.at[buf], gp_v.at[buf], phw_v.at[buf], php_v.at[buf]
        pltpu.make_async_copy(word_hbm.at[idxw_v.at[buf]], gw, semw.at[buf]).wait()
        pltpu.make_async_copy(posemb_hbm.at[idxp_v.at[buf]], gp, semp.at[buf]).wait()
        tid_s[...] = plsc.load_gather(tidq_v, [c * K + lo8])

        hoist = []
        for t in range(K):
            t16 = jnp.full((L,), t + 8, jnp.int32)
            phw = plsc.load_gather(phw_b, [t16])
            php = plsc.load_gather(php_b, [t16])
            tidf = plsc.load_gather(tid_s, [t16]).astype(jnp.float32)
            aw = phw - 1 + iota
            ap = php - 1 + iota
            hoist.append((8 * (aw >> 4) + t, aw & 15,
                          8 * (ap >> 4) + t, ap & 15, tidf, phw, php))

        def col_body(j, sumsqs):
            j8 = j * 8
            cvec = j * L - 1 + iota
            ty0c = plsc.load_gather(ty_v, [cvec])
            dtyc = plsc.load_gather(dty_v, [cvec])
            out = []
            for t in range(K):
                rw, lw, rp, lp, tidf, _, _ = hoist[t]
                x = (plsc.load_gather(gw, [rw + j8, lw])
                     + plsc.load_gather(gp, [rp + j8, lp]))
                x = x + ty0c + tidf * dtyc
                obuf[t, pl.ds(j * L, L)] = x
                out.append(sumsqs[t] + x * x)
            return tuple(out)

        sumsqs = lax.fori_loop(1, njc, col_body,
                               tuple(jnp.zeros((L,), jnp.float32)
                                     for _ in range(K)))
        c0 = jnp.maximum(iota - 1, 0)
        ty00 = plsc.load_gather(ty_v, [c0])
        dty0 = plsc.load_gather(dty_v, [c0])
        for t in range(K):
            _, _, _, _, tidf, phw, php = hoist[t]
            q0w = phw + c0
            q0p = php + c0
            x0 = (plsc.load_gather(gw, [8 * (q0w >> 4) + t, q0w & 15])
                  + plsc.load_gather(gp, [8 * (q0p >> 4) + t, q0p & 15]))
            x0 = x0 + ty00 + tidf * dty0
            ssq = sumsqs[t] + jnp.where(iota >= 1, x0 * x0, 0.0)
            s = inv_c + jnp.sum(ssq)
            time = s * _rsqrt_newton(s)
            obuf[t, pl.ds(0, L)] = jnp.where(iota == 0, time, x0)

        pltpu.sync_copy(obuf, out_hbm.at[pl.ds(base + c * K, K)])

    launch(0, 0)

    def pair_body(i, _):
        c0 = 2 * i
        launch(c0 + 1, 1)
        compute(c0, 0)

        @pl.when(i < npair - 1)
        def _():
            launch(c0 + 2, 0)

        compute(c0 + 1, 1)
        return 0

    lax.fori_loop(0, npair, pair_body, 0)


def kernel(input_ids, position_ids, token_type_ids, word_emb, pos_emb,
           type_emb, curvature):
    B, S = input_ids.shape
    V, Dm1 = word_emb.shape
    MP = pos_emb.shape[0]
    N = B * S
    D = Dm1 + 1
    NBw = V * Dm1 // L
    NBp = MP * Dm1 // L

    ids = input_ids.reshape(-1).astype(jnp.int32)
    pos = position_ids.reshape(-1).astype(jnp.int32)
    tid = token_type_ids.reshape(-1).astype(jnp.int32)
    wblk = word_emb.reshape(NBw, L)
    pblk = pos_emb.reshape(NBp, L)
    tyflat = type_emb.reshape(-1)
    curv16 = jnp.full((L,), curvature, jnp.float32)

    mesh = plsc.VectorSubcoreMesh(core_axis_name="c", subcore_axis_name="s",
                                  num_cores=NC, num_subcores=NS)
    body = functools.partial(_sc_body, N, Dm1, NBw, NBp)
    sck = pl.kernel(
        body,
        out_type=jax.ShapeDtypeStruct((N, D), jnp.float32),
        mesh=mesh,
        compiler_params=pltpu.CompilerParams(use_tc_tiling_on_sc=False,
                                             needs_layout_passes=False),
        scratch_types=[
            pltpu.VMEM((N // NW,), jnp.int32),
            pltpu.VMEM((N // NW,), jnp.int32),
            pltpu.VMEM((N // NW,), jnp.int32),
            pltpu.VMEM((L,), jnp.int32),
            pltpu.VMEM((2, L), jnp.int32),
            pltpu.VMEM((2, L), jnp.int32),
            pltpu.VMEM((2, NI), jnp.int32),
            pltpu.VMEM((2, NI), jnp.int32),
            pltpu.VMEM((2, NI, L), jnp.float32),
            pltpu.VMEM((2, NI, L), jnp.float32),
            pltpu.VMEM((2 * D,), jnp.float32),
            pltpu.VMEM((D,), jnp.float32),
            pltpu.VMEM((K, D), jnp.float32),
            pltpu.VMEM((L,), jnp.float32),
            pltpu.SemaphoreType.DMA((2,)),
            pltpu.SemaphoreType.DMA((2,)),
        ],
    )
    out = sck(ids, pos, tid, wblk, pblk, tyflat, curv16)
    return out.reshape(B, S, D)


# speedup vs baseline: 1.7430x; 1.4064x over previous
import functools

import jax
import jax.numpy as jnp
from jax import lax
from jax.experimental import pallas as pl
from jax.experimental.pallas import tpu as pltpu
from jax.experimental.pallas import tpu_sc as plsc

NC = 2
NS = 16
L = 16
NW = NC * NS

K = 8
R = 129
NI = R * K
MAGIC = 0x5F3759DF


def _rsqrt_newton(s):
    i = plsc.bitcast(s, jnp.int32)
    i = jnp.int32(MAGIC) - (i >> 1)
    y = plsc.bitcast(i, jnp.float32)
    for _ in range(3):
        y = y * (1.5 - 0.5 * s * y * y)
    return y


def _sc_body(N, Dm1, NBw, NBp,
             ids_hbm, pos_hbm, tid_hbm, word_hbm, posemb_hbm, type_hbm,
             curv_hbm, out_hbm,
             ids_v, pos_v, tidq_v, tid_s, phw_v, php_v, idxw_v, idxp_v,
             gw_v, gp_v, ty_v, dty_v, obuf, curv_v, semw, semp):
    D = Dm1 + 1
    tpw = N // NW
    nchunk = tpw // K
    npair = nchunk // 2
    njc = D // L
    wid = lax.axis_index("s") * NC + lax.axis_index("c")
    base = wid * tpw

    iota = lax.iota(jnp.int32, L)
    lo8 = iota & 7
    hi8 = jnp.where(iota >= 8, 1, 0)

    pltpu.sync_copy(curv_hbm, curv_v)
    inv_c = 1.0 / curv_v[...]

    pltpu.sync_copy(ids_hbm.at[pl.ds(base, tpw)], ids_v)
    pltpu.sync_copy(pos_hbm.at[pl.ds(base, tpw)], pos_v)
    pltpu.sync_copy(tid_hbm.at[pl.ds(base, tpw)], tidq_v)

    pltpu.sync_copy(type_hbm, ty_v.at[pl.ds(0, 2 * Dm1)])
    for jj in range(njc):
        cidx = jnp.minimum(jj * L + iota, Dm1 - 1)
        d = (plsc.load_gather(ty_v, [Dm1 + cidx])
             - plsc.load_gather(ty_v, [cidx]))
        dty_v[pl.ds(jj * L, L)] = d

    def launch(c, buf):
        idxw, idxp, phw_b, php_b = idxw_v.at[buf], idxp_v.at[buf], phw_v.at[buf], php_v.at[buf]
        sel = c * K + lo8
        idsd = plsc.load_gather(ids_v, [sel])
        posd = plsc.load_gather(pos_v, [sel])
        sw = idsd * Dm1
        sp = posd * Dm1
        b0w = sw >> 4
        b0p = sp >> 4
        phw_b[...] = sw & 15
        php_b[...] = sp & 15
        for i in range(64):
            idxw[pl.ds(16 * i, L)] = b0w + (2 * i + hi8)
            idxp[pl.ds(16 * i, L)] = b0p + (2 * i + hi8)
        plsc.store_scatter(idxw, [1024 + iota],
                           jnp.minimum(b0w + 128, NBw - 1), mask=iota < 8)
        plsc.store_scatter(idxp, [1024 + iota],
                           jnp.minimum(b0p + 128, NBp - 1), mask=iota < 8)
        pltpu.async_copy(word_hbm.at[idxw], gw_v.at[buf], semw.at[buf])
        pltpu.async_copy(posemb_hbm.at[idxp], gp_v.at[buf], semp.at[buf])

    def compute(c, buf):
        gw, gp, phw_b, php_b = gw_v.at[buf], gp_v.at[buf], phw_v.at[buf], php_v.at[buf]
        pltpu.make_async_copy(word_hbm.at[idxw_v.at[buf]], gw, semw.at[buf]).wait()
        pltpu.make_async_copy(posemb_hbm.at[idxp_v.at[buf]], gp, semp.at[buf]).wait()
        tid_s[...] = plsc.load_gather(tidq_v, [c * K + lo8])

        hoist = []
        for t in range(K):
            t16 = jnp.full((L,), t + 8, jnp.int32)
            phw = plsc.load_gather(phw_b, [t16])
            php = plsc.load_gather(php_b, [t16])
            tidf = plsc.load_gather(tid_s, [t16]).astype(jnp.float32)
            aw = phw - 1 + iota
            ap = php - 1 + iota
            hoist.append((8 * (aw >> 4) + t, aw & 15,
                          8 * (ap >> 4) + t, ap & 15, tidf, phw, php))

        sumsqs = [None] * K

        for g in range(0, K, 4):
            def col_step(j, sq, g=g):
                j8 = j * 8
                cvec = j * L - 1 + iota
                ty0c = plsc.load_gather(ty_v, [cvec])
                dtyc = plsc.load_gather(dty_v, [cvec])
                out = []
                for u, t in enumerate(range(g, g + 4)):
                    rw, lw, rp, lp, tidf, _, _ = hoist[t]
                    x = (plsc.load_gather(gw, [rw + j8, lw])
                         + plsc.load_gather(gp, [rp + j8, lp]))
                    x = x + ty0c + tidf * dtyc
                    obuf[t, pl.ds(j * L, L)] = x
                    out.append(sq[u] + x * x)
                return tuple(out)

            init = tuple(jnp.zeros((L,), jnp.float32) for _ in range(4))
            res = plsc.parallel_loop(1, njc - 1, 1, unroll=6,
                                     carry=init)(col_step)
            res = col_step(njc - 1, res)
            for u, t in enumerate(range(g, g + 4)):
                sumsqs[t] = res[u]
        c0 = jnp.maximum(iota - 1, 0)
        ty00 = plsc.load_gather(ty_v, [c0])
        dty0 = plsc.load_gather(dty_v, [c0])
        for t in range(K):
            _, _, _, _, tidf, phw, php = hoist[t]
            q0w = phw + c0
            q0p = php + c0
            x0 = (plsc.load_gather(gw, [8 * (q0w >> 4) + t, q0w & 15])
                  + plsc.load_gather(gp, [8 * (q0p >> 4) + t, q0p & 15]))
            x0 = x0 + ty00 + tidf * dty0
            ssq = sumsqs[t] + jnp.where(iota >= 1, x0 * x0, 0.0)
            s = inv_c + jnp.sum(ssq)
            time = s * _rsqrt_newton(s)
            obuf[t, pl.ds(0, L)] = jnp.where(iota == 0, time, x0)

        pltpu.sync_copy(obuf, out_hbm.at[pl.ds(base + c * K, K)])

    launch(0, 0)

    def pair_body(i, _):
        c0 = 2 * i
        launch(c0 + 1, 1)
        compute(c0, 0)

        @pl.when(i < npair - 1)
        def _():
            launch(c0 + 2, 0)

        compute(c0 + 1, 1)
        return 0

    lax.fori_loop(0, npair, pair_body, 0)


def kernel(input_ids, position_ids, token_type_ids, word_emb, pos_emb,
           type_emb, curvature):
    B, S = input_ids.shape
    V, Dm1 = word_emb.shape
    MP = pos_emb.shape[0]
    N = B * S
    D = Dm1 + 1
    NBw = V * Dm1 // L
    NBp = MP * Dm1 // L

    ids = input_ids.reshape(-1).astype(jnp.int32)
    pos = position_ids.reshape(-1).astype(jnp.int32)
    tid = token_type_ids.reshape(-1).astype(jnp.int32)
    wblk = word_emb.reshape(NBw, L)
    pblk = pos_emb.reshape(NBp, L)
    tyflat = type_emb.reshape(-1)
    curv16 = jnp.full((L,), curvature, jnp.float32)

    mesh = plsc.VectorSubcoreMesh(core_axis_name="c", subcore_axis_name="s",
                                  num_cores=NC, num_subcores=NS)
    body = functools.partial(_sc_body, N, Dm1, NBw, NBp)
    sck = pl.kernel(
        body,
        out_type=jax.ShapeDtypeStruct((N, D), jnp.float32),
        mesh=mesh,
        compiler_params=pltpu.CompilerParams(use_tc_tiling_on_sc=False,
                                             needs_layout_passes=False),
        scratch_types=[
            pltpu.VMEM((N // NW,), jnp.int32),
            pltpu.VMEM((N // NW,), jnp.int32),
            pltpu.VMEM((N // NW,), jnp.int32),
            pltpu.VMEM((L,), jnp.int32),
            pltpu.VMEM((2, L), jnp.int32),
            pltpu.VMEM((2, L), jnp.int32),
            pltpu.VMEM((2, NI), jnp.int32),
            pltpu.VMEM((2, NI), jnp.int32),
            pltpu.VMEM((2, NI, L), jnp.float32),
            pltpu.VMEM((2, NI, L), jnp.float32),
            pltpu.VMEM((2 * D,), jnp.float32),
            pltpu.VMEM((D,), jnp.float32),
            pltpu.VMEM((K, D), jnp.float32),
            pltpu.VMEM((L,), jnp.float32),
            pltpu.SemaphoreType.DMA((2,)),
            pltpu.SemaphoreType.DMA((2,)),
        ],
    )
    out = sck(ids, pos, tid, wblk, pblk, tyflat, curv16)
    return out.reshape(B, S, D)
